# Initial kernel scaffold; baseline (speedup 1.0000x reference)
import jax
import jax.numpy as jnp
from jax.experimental import pallas as pl


def kernel(input_ids, word_emb, pos_emb, tok_emb, ln_gamma, ln_beta):
    raise NotImplementedError("write your pallas kernel here")


# speedup vs baseline: 1.6023x; 1.6023x over previous
import functools

import jax
import jax.numpy as jnp
from jax import lax
from jax.experimental import pallas as pl
from jax.experimental.pallas import tpu as pltpu
from jax.experimental.pallas import tpu_sc as plsc

_VOCAB = 100000
_MAX_POS = 2048
_HIDDEN = 1024
_BATCH = 4
_SEQ = 2048
_EPS = 1e-06

_NC = 2
_NS = 16
_NW = _NC * _NS
_B = _BATCH * _SEQ
_BPW = _B // _NW
_CH = 32
_NCHUNK = _BPW // _CH

_mesh = plsc.VectorSubcoreMesh(core_axis_name="c", subcore_axis_name="s")


@functools.partial(
    pl.kernel,
    mesh=_mesh,
    out_type=jax.ShapeDtypeStruct((_B, _HIDDEN), jnp.float32),
    scratch_types=[
        pltpu.VMEM((_NCHUNK, _CH), jnp.int32),
        pltpu.VMEM((_CH, _HIDDEN), jnp.float32),
        pltpu.VMEM((_CH, _HIDDEN), jnp.float32),
        pltpu.SemaphoreType.DMA,
        pltpu.SemaphoreType.DMA,
        pltpu.SemaphoreType.DMA,
        pltpu.SemaphoreType.DMA,
    ],
)
def _sc_gather(idx_hbm, table_hbm, out_hbm, idx_v, buf0, buf1, g0, g1, o0, o1):
    wid = lax.axis_index("s") * _NC + lax.axis_index("c")
    base = wid * _BPW
    bufs = (buf0, buf1)
    gsems = (g0, g1)
    osems = (o0, o1)
    pltpu.sync_copy(idx_hbm.at[wid], idx_v)
    gathers = [None] * _NCHUNK
    outs = [None] * _NCHUNK
    gathers[0] = pltpu.async_copy(table_hbm.at[idx_v.at[0]], bufs[0], gsems[0])
    for c in range(_NCHUNK):
        b = c % 2
        nb = (c + 1) % 2
        if c + 1 < _NCHUNK:
            if c >= 1:
                outs[c - 1].wait()
            gathers[c + 1] = pltpu.async_copy(
                table_hbm.at[idx_v.at[c + 1]], bufs[nb], gsems[nb])
        gathers[c].wait()
        outs[c] = pltpu.async_copy(
            bufs[b], out_hbm.at[pl.ds(base + c * _CH, _CH)], osems[b])
    outs[_NCHUNK - 1].wait()


def _ln_body(x_ref, pos_ref, tok_ref, gamma_ref, beta_ref, out_ref):
    x = x_ref[...] + pos_ref[...] + tok_ref[...]
    mean = jnp.mean(x, axis=-1, keepdims=True)
    xc = x - mean
    var = jnp.mean(xc * xc, axis=-1, keepdims=True)
    out_ref[...] = xc * lax.rsqrt(var + _EPS) * gamma_ref[...] + beta_ref[...]


_LN_ROWS = 256


def _tc_layernorm(gathered, pos_emb, tok_row, gamma, beta):
    grid = (_B // _LN_ROWS,)
    blocks_per_seq = _SEQ // _LN_ROWS
    return pl.pallas_call(
        _ln_body,
        grid=grid,
        in_specs=[
            pl.BlockSpec((_LN_ROWS, _HIDDEN), lambda i: (i, 0)),
            pl.BlockSpec((_LN_ROWS, _HIDDEN),
                         lambda i: (i % blocks_per_seq, 0)),
            pl.BlockSpec((1, _HIDDEN), lambda i: (0, 0)),
            pl.BlockSpec((1, _HIDDEN), lambda i: (0, 0)),
            pl.BlockSpec((1, _HIDDEN), lambda i: (0, 0)),
        ],
        out_specs=pl.BlockSpec((_LN_ROWS, _HIDDEN), lambda i: (i, 0)),
        out_shape=jax.ShapeDtypeStruct((_B, _HIDDEN), jnp.float32),
        compiler_params=pltpu.CompilerParams(
            dimension_semantics=("arbitrary",)),
    )(gathered, pos_emb, tok_row, gamma, beta)


def kernel(input_ids, word_emb, pos_emb, tok_emb, ln_gamma, ln_beta):
    ids = input_ids.reshape(_NW, _NCHUNK, _CH).astype(jnp.int32)
    gathered = _sc_gather(ids, word_emb)
    out = _tc_layernorm(
        gathered,
        pos_emb[:_SEQ],
        tok_emb[0:1],
        ln_gamma.reshape(1, _HIDDEN),
        ln_beta.reshape(1, _HIDDEN),
    )
    return out.reshape(_BATCH, _SEQ, _HIDDEN)


# speedup vs baseline: 1.6261x; 1.0149x over previous
import functools

import jax
import jax.numpy as jnp
from jax import lax
from jax.experimental import pallas as pl
from jax.experimental.pallas import tpu as pltpu
from jax.experimental.pallas import tpu_sc as plsc

_VOCAB = 100000
_MAX_POS = 2048
_HIDDEN = 1024
_BATCH = 4
_SEQ = 2048
_EPS = 1e-06

_NC = 2
_NS = 16
_NW = _NC * _NS
_B = _BATCH * _SEQ
_BPW = _B // _NW
_CH = 32
_NCHUNK = _BPW // _CH

_mesh = plsc.VectorSubcoreMesh(core_axis_name="c", subcore_axis_name="s")


@functools.partial(
    pl.kernel,
    mesh=_mesh,
    out_type=jax.ShapeDtypeStruct((_B, _HIDDEN), jnp.float32),
    scratch_types=[
        pltpu.VMEM((_NCHUNK, _CH), jnp.int32),
        pltpu.VMEM((_CH, _HIDDEN), jnp.float32),
        pltpu.VMEM((_CH, _HIDDEN), jnp.float32),
        pltpu.SemaphoreType.DMA,
        pltpu.SemaphoreType.DMA,
        pltpu.SemaphoreType.DMA,
        pltpu.SemaphoreType.DMA,
    ],
)
def _sc_gather(idx_hbm, table_hbm, out_hbm, idx_v, buf0, buf1, g0, g1, o0, o1):
    wid = lax.axis_index("s") * _NC + lax.axis_index("c")
    base = wid * _BPW
    bufs = (buf0, buf1)
    gsems = (g0, g1)
    osems = (o0, o1)
    pltpu.sync_copy(idx_hbm.at[wid], idx_v)
    gathers = [None] * _NCHUNK
    outs = [None] * _NCHUNK
    gathers[0] = pltpu.async_copy(table_hbm.at[idx_v.at[0]], bufs[0], gsems[0])
    for c in range(_NCHUNK):
        b = c % 2
        nb = (c + 1) % 2
        if c + 1 < _NCHUNK:
            if c >= 1:
                outs[c - 1].wait()
            gathers[c + 1] = pltpu.async_copy(
                table_hbm.at[idx_v.at[c + 1]], bufs[nb], gsems[nb])
        gathers[c].wait()
        outs[c] = pltpu.async_copy(
            bufs[b], out_hbm.at[pl.ds(base + c * _CH, _CH)], osems[b])
    outs[_NCHUNK - 1].wait()


def _ln_body(x_ref, pos_ref, tok_ref, gamma_ref, beta_ref, out_ref):
    x = x_ref[...] + pos_ref[...] + tok_ref[...]
    mean = jnp.mean(x, axis=-1, keepdims=True)
    xc = x - mean
    var = jnp.mean(xc * xc, axis=-1, keepdims=True)
    out_ref[...] = xc * lax.rsqrt(var + _EPS) * gamma_ref[...] + beta_ref[...]


_LN_ROWS = 256


def _tc_layernorm(gathered, pos_emb, tok_row, gamma, beta):
    blocks_per_seq = _SEQ // _LN_ROWS
    grid = (blocks_per_seq, _BATCH)
    return pl.pallas_call(
        _ln_body,
        grid=grid,
        in_specs=[
            pl.BlockSpec((_LN_ROWS, _HIDDEN),
                         lambda i, b: (b * blocks_per_seq + i, 0)),
            pl.BlockSpec((_LN_ROWS, _HIDDEN), lambda i, b: (i, 0)),
            pl.BlockSpec((1, _HIDDEN), lambda i, b: (0, 0)),
            pl.BlockSpec((1, _HIDDEN), lambda i, b: (0, 0)),
            pl.BlockSpec((1, _HIDDEN), lambda i, b: (0, 0)),
        ],
        out_specs=pl.BlockSpec((_LN_ROWS, _HIDDEN),
                               lambda i, b: (b * blocks_per_seq + i, 0)),
        out_shape=jax.ShapeDtypeStruct((_B, _HIDDEN), jnp.float32),
        compiler_params=pltpu.CompilerParams(
            dimension_semantics=("arbitrary", "arbitrary")),
    )(gathered, pos_emb, tok_row, gamma, beta)


def kernel(input_ids, word_emb, pos_emb, tok_emb, ln_gamma, ln_beta):
    ids = input_ids.reshape(_NW, _NCHUNK, _CH).astype(jnp.int32)
    gathered = _sc_gather(ids, word_emb)
    out = _tc_layernorm(
        gathered,
        pos_emb[:_SEQ],
        tok_emb[0:1],
        ln_gamma.reshape(1, _HIDDEN),
        ln_beta.reshape(1, _HIDDEN),
    )
    return out.reshape(_BATCH, _SEQ, _HIDDEN)
